# baseline (speedup 1.0000x reference)
import functools
import jax
import jax.numpy as jnp
from jax.experimental import pallas as pl
from jax.experimental.pallas import tpu as pltpu

N = 2048
D = 1024
E = 8
R = 16
H = 256
K = 2

BN = 256
GRID = N // BN


def _body(x_ref, wg1_ref, bg1_ref, wg2_ref, bg2_ref, wf_ref, a_ref, b_ref,
          bias_ref, out_ref, loss_ref, acc_m, acc_p):
    step = pl.program_id(0)
    xb = x_ref[...]

    h = jax.lax.dot_general(xb, wg1_ref[...], (((1,), (1,)), ((), ())),
                            preferred_element_type=jnp.float32)
    h = jnp.maximum(h + bg1_ref[...], 0.0)
    logits = jax.lax.dot_general(h, wg2_ref[...], (((1,), (1,)), ((), ())),
                                 preferred_element_type=jnp.float32)
    logits = logits + bg2_ref[...]

    neg = jnp.float32(-jnp.inf)
    m1 = jnp.full((BN, 1), neg, jnp.float32)
    i1 = jnp.zeros((BN, 1), jnp.int32)
    for e in range(E):
        v = logits[:, e:e + 1]
        take = v > m1
        m1 = jnp.where(take, v, m1)
        i1 = jnp.where(take, e, i1)
    m2 = jnp.full((BN, 1), neg, jnp.float32)
    i2 = jnp.zeros((BN, 1), jnp.int32)
    for e in range(E):
        v = logits[:, e:e + 1]
        take = (v > m2) & (i1 != e)
        m2 = jnp.where(take, v, m2)
        i2 = jnp.where(take, e, i2)

    a = jnp.exp(m2 - m1)
    g1 = 1.0 / (1.0 + a)
    g2 = 1.0 - g1

    eiota = jax.lax.broadcasted_iota(jnp.int32, (BN, E), 1)
    sel1 = eiota == i1
    sel2 = eiota == i2
    w = jnp.where(sel1, g1, 0.0) + jnp.where(sel2, g2, 0.0)
    mask = (sel1 | sel2).astype(jnp.float32)

    ex = jnp.exp(logits - m1)
    probs = ex / jnp.sum(ex, axis=1, keepdims=True)

    xb16 = xb.astype(jnp.bfloat16)
    of = jax.lax.dot_general(xb16, wf_ref[...], (((1,), (1,)), ((), ())),
                             preferred_element_type=jnp.float32)
    xa = jax.lax.dot_general(xb16, a_ref[...], (((1,), (1,)), ((), ())),
                             preferred_element_type=jnp.float32)

    rows = jax.lax.broadcasted_iota(jnp.int32, (E, E * R), 0)
    cols = jax.lax.broadcasted_iota(jnp.int32, (E, E * R), 1)
    sel = (cols // R == rows).astype(jnp.float32)
    w_wide = jax.lax.dot_general(w, sel, (((1,), (0,)), ((), ())),
                                 preferred_element_type=jnp.float32)

    xaw = (xa * w_wide).astype(jnp.bfloat16)
    eo = jax.lax.dot_general(xaw, b_ref[...], (((1,), (0,)), ((), ())),
                             preferred_element_type=jnp.float32)
    out_ref[...] = of + eo + bias_ref[...]

    bm = jnp.sum(mask, axis=0, keepdims=True)
    bp = jnp.sum(probs, axis=0, keepdims=True)

    @pl.when(step == 0)
    def _():
        acc_m[...] = bm
        acc_p[...] = bp

    @pl.when(step > 0)
    def _():
        acc_m[...] += bm
        acc_p[...] += bp

    @pl.when(step == GRID - 1)
    def _():
        loss = (E / (N * N)) * jnp.sum(acc_m[...] * acc_p[...], keepdims=True)
        loss_ref[...] = loss.reshape(1, 1)


@jax.jit
def _run(x, Wg1, bg1, Wg2, bg2, weight_fixed, A_flat, B_flat, bias):
    full = lambda s: pl.BlockSpec(s, lambda i: (0, 0))
    out, loss = pl.pallas_call(
        _body,
        grid=(GRID,),
        in_specs=[
            pl.BlockSpec((BN, D), lambda i: (i, 0)),
            full((H, D)),
            full((1, H)),
            full((E, H)),
            full((1, E)),
            full((D, D)),
            full((E * R, D)),
            full((E * R, D)),
            full((1, D)),
        ],
        out_specs=[
            pl.BlockSpec((BN, D), lambda i: (i, 0)),
            full((1, 1)),
        ],
        out_shape=[
            jax.ShapeDtypeStruct((N, D), jnp.float32),
            jax.ShapeDtypeStruct((1, 1), jnp.float32),
        ],
        scratch_shapes=[
            pltpu.VMEM((1, E), jnp.float32),
            pltpu.VMEM((1, E), jnp.float32),
        ],
        compiler_params=pltpu.CompilerParams(
            dimension_semantics=("arbitrary",),
        ),
    )(x, Wg1, bg1.reshape(1, H), Wg2, bg2.reshape(1, E),
      weight_fixed.astype(jnp.bfloat16), A_flat.astype(jnp.bfloat16),
      B_flat.astype(jnp.bfloat16), bias.reshape(1, D))
    return out, loss[0, 0]


def kernel(x, Wg1, bg1, Wg2, bg2, weight_fixed, A_all, B_all, bias):
    A_flat = A_all.reshape(E * R, D)
    B_flat = B_all.transpose(0, 2, 1).reshape(E * R, D)
    return _run(x, Wg1, bg1, Wg2, bg2, weight_fixed, A_flat, B_flat, bias)


# speedup vs baseline: 1.2245x; 1.2245x over previous
import functools
import jax
import jax.numpy as jnp
from jax.experimental import pallas as pl
from jax.experimental.pallas import tpu as pltpu

N = 2048
D = 1024
E = 8
R = 16
H = 256
K = 2

BN = 256
GRID = N // BN


def _body(x_ref, wg1_ref, bg1_ref, wg2_ref, bg2_ref, wf_ref, a_ref, b_ref,
          bias_ref, out_ref, loss_ref, acc_m, acc_p):
    step = pl.program_id(0)
    xb = x_ref[...]

    h = jax.lax.dot_general(xb, wg1_ref[...], (((1,), (1,)), ((), ())),
                            preferred_element_type=jnp.float32)
    h = jnp.maximum(h + bg1_ref[...], 0.0)
    logits = jax.lax.dot_general(h, wg2_ref[...], (((1,), (1,)), ((), ())),
                                 preferred_element_type=jnp.float32)
    logits = logits + bg2_ref[...]

    neg = jnp.float32(-jnp.inf)
    m1 = jnp.full((BN, 1), neg, jnp.float32)
    i1 = jnp.zeros((BN, 1), jnp.int32)
    for e in range(E):
        v = logits[:, e:e + 1]
        take = v > m1
        m1 = jnp.where(take, v, m1)
        i1 = jnp.where(take, e, i1)
    m2 = jnp.full((BN, 1), neg, jnp.float32)
    i2 = jnp.zeros((BN, 1), jnp.int32)
    for e in range(E):
        v = logits[:, e:e + 1]
        take = (v > m2) & (i1 != e)
        m2 = jnp.where(take, v, m2)
        i2 = jnp.where(take, e, i2)

    a = jnp.exp(m2 - m1)
    g1 = 1.0 / (1.0 + a)
    g2 = 1.0 - g1

    eiota = jax.lax.broadcasted_iota(jnp.int32, (BN, E), 1)
    sel1 = eiota == i1
    sel2 = eiota == i2
    w = jnp.where(sel1, g1, 0.0) + jnp.where(sel2, g2, 0.0)
    mask = (sel1 | sel2).astype(jnp.float32)

    ex = jnp.exp(logits - m1)
    probs = ex / jnp.sum(ex, axis=1, keepdims=True)

    of = jax.lax.dot_general(xb, wf_ref[...], (((1,), (1,)), ((), ())),
                             preferred_element_type=jnp.float32)
    xa = jax.lax.dot_general(xb, a_ref[...], (((1,), (1,)), ((), ())),
                             preferred_element_type=jnp.float32)

    rows = jax.lax.broadcasted_iota(jnp.int32, (E, E * R), 0)
    cols = jax.lax.broadcasted_iota(jnp.int32, (E, E * R), 1)
    sel = (cols // R == rows).astype(jnp.float32)
    w_wide = jax.lax.dot_general(w, sel, (((1,), (0,)), ((), ())),
                                 preferred_element_type=jnp.float32)

    eo = jax.lax.dot_general(xa * w_wide, b_ref[...], (((1,), (0,)), ((), ())),
                             preferred_element_type=jnp.float32)
    out_ref[...] = of + eo + bias_ref[...]

    bm = jnp.sum(mask, axis=0, keepdims=True)
    bp = jnp.sum(probs, axis=0, keepdims=True)

    @pl.when(step == 0)
    def _():
        acc_m[...] = bm
        acc_p[...] = bp

    @pl.when(step > 0)
    def _():
        acc_m[...] += bm
        acc_p[...] += bp

    @pl.when(step == GRID - 1)
    def _():
        loss = (E / (N * N)) * jnp.sum(acc_m[...] * acc_p[...], keepdims=True)
        loss_ref[...] = loss.reshape(1, 1)


@jax.jit
def _run(x, Wg1, bg1, Wg2, bg2, weight_fixed, A_flat, B_flat, bias):
    full = lambda s: pl.BlockSpec(s, lambda i: (0, 0))
    out, loss = pl.pallas_call(
        _body,
        grid=(GRID,),
        in_specs=[
            pl.BlockSpec((BN, D), lambda i: (i, 0)),
            full((H, D)),
            full((1, H)),
            full((E, H)),
            full((1, E)),
            full((D, D)),
            full((E * R, D)),
            full((E * R, D)),
            full((1, D)),
        ],
        out_specs=[
            pl.BlockSpec((BN, D), lambda i: (i, 0)),
            full((1, 1)),
        ],
        out_shape=[
            jax.ShapeDtypeStruct((N, D), jnp.float32),
            jax.ShapeDtypeStruct((1, 1), jnp.float32),
        ],
        scratch_shapes=[
            pltpu.VMEM((1, E), jnp.float32),
            pltpu.VMEM((1, E), jnp.float32),
        ],
        compiler_params=pltpu.CompilerParams(
            dimension_semantics=("arbitrary",),
        ),
    )(x, Wg1, bg1.reshape(1, H), Wg2, bg2.reshape(1, E),
      weight_fixed, A_flat, B_flat, bias.reshape(1, D))
    return out, loss[0, 0]


def kernel(x, Wg1, bg1, Wg2, bg2, weight_fixed, A_all, B_all, bias):
    A_flat = A_all.reshape(E * R, D)
    B_flat = B_all.transpose(0, 2, 1).reshape(E * R, D)
    return _run(x, Wg1, bg1, Wg2, bg2, weight_fixed, A_flat, B_flat, bias)


# speedup vs baseline: 1.8036x; 1.4729x over previous
import jax
import jax.numpy as jnp
from jax.experimental import pallas as pl
from jax.experimental.pallas import tpu as pltpu

N = 2048
D = 1024
E = 8
R = 16
H = 256
K = 2

BN = 256
GRID = N // BN


def _body(x_ref, wg1_ref, bg1_ref, wg2_ref, bg2_ref, wf_ref, a_ref, b_ref,
          bias_ref, out_ref, loss_ref, acc_m, acc_p):
    step = pl.program_id(0)
    xb = x_ref[...]

    h = jax.lax.dot_general(xb, wg1_ref[...], (((1,), (1,)), ((), ())),
                            preferred_element_type=jnp.float32)
    h = jnp.maximum(h + bg1_ref[...], 0.0)
    logits = jax.lax.dot_general(wg2_ref[...], h, (((1,), (1,)), ((), ())),
                                 preferred_element_type=jnp.float32)
    logits = logits + bg2_ref[...]

    neg = jnp.float32(-jnp.inf)
    m1 = jnp.full((1, BN), neg, jnp.float32)
    i1 = jnp.zeros((1, BN), jnp.int32)
    for e in range(E):
        v = logits[e:e + 1, :]
        take = v > m1
        m1 = jnp.where(take, v, m1)
        i1 = jnp.where(take, e, i1)
    m2 = jnp.full((1, BN), neg, jnp.float32)
    i2 = jnp.zeros((1, BN), jnp.int32)
    for e in range(E):
        v = logits[e:e + 1, :]
        take = (v > m2) & (i1 != e)
        m2 = jnp.where(take, v, m2)
        i2 = jnp.where(take, e, i2)

    a = jnp.exp(m2 - m1)
    g1 = 1.0 / (1.0 + a)
    g2 = 1.0 - g1

    eiota = jax.lax.broadcasted_iota(jnp.int32, (E, BN), 0)
    sel1 = eiota == i1
    sel2 = eiota == i2
    wt = jnp.where(sel1, g1, 0.0) + jnp.where(sel2, g2, 0.0)
    maskt = (sel1 | sel2).astype(jnp.float32)

    ex = jnp.exp(logits - m1)
    probst = ex / jnp.sum(ex, axis=0, keepdims=True)

    of = jax.lax.dot_general(xb, wf_ref[...], (((1,), (1,)), ((), ())),
                             preferred_element_type=jnp.float32)
    xa = jax.lax.dot_general(xb, a_ref[...], (((1,), (1,)), ((), ())),
                             preferred_element_type=jnp.float32)

    rows = jax.lax.broadcasted_iota(jnp.int32, (E, E * R), 0)
    cols = jax.lax.broadcasted_iota(jnp.int32, (E, E * R), 1)
    sel = (cols // R == rows).astype(jnp.float32)
    w_wide = jax.lax.dot_general(wt, sel, (((0,), (0,)), ((), ())),
                                 preferred_element_type=jnp.float32)

    eo = jax.lax.dot_general(xa * w_wide, b_ref[...], (((1,), (0,)), ((), ())),
                             preferred_element_type=jnp.float32)
    out_ref[...] = of + eo + bias_ref[...]

    @pl.when(step == 0)
    def _():
        acc_m[...] = maskt
        acc_p[...] = probst

    @pl.when(step > 0)
    def _():
        acc_m[...] += maskt
        acc_p[...] += probst

    @pl.when(step == GRID - 1)
    def _():
        am = jnp.sum(acc_m[...], axis=1, keepdims=True)
        ap = jnp.sum(acc_p[...], axis=1, keepdims=True)
        loss = (E / (N * N)) * jnp.sum(am * ap, keepdims=True)
        loss_ref[...] = loss.reshape(1, 1)


@jax.jit
def _run(x, Wg1, bg1, Wg2, bg2, weight_fixed, A_flat, B_flat, bias):
    full = lambda s: pl.BlockSpec(s, lambda i: (0, 0))
    out, loss = pl.pallas_call(
        _body,
        grid=(GRID,),
        in_specs=[
            pl.BlockSpec((BN, D), lambda i: (i, 0)),
            full((H, D)),
            full((1, H)),
            full((E, H)),
            full((E, 1)),
            full((D, D)),
            full((E * R, D)),
            full((E * R, D)),
            full((1, D)),
        ],
        out_specs=[
            pl.BlockSpec((BN, D), lambda i: (i, 0)),
            full((1, 1)),
        ],
        out_shape=[
            jax.ShapeDtypeStruct((N, D), jnp.float32),
            jax.ShapeDtypeStruct((1, 1), jnp.float32),
        ],
        scratch_shapes=[
            pltpu.VMEM((E, BN), jnp.float32),
            pltpu.VMEM((E, BN), jnp.float32),
        ],
        compiler_params=pltpu.CompilerParams(
            dimension_semantics=("arbitrary",),
        ),
    )(x, Wg1, bg1.reshape(1, H), Wg2, bg2.reshape(E, 1),
      weight_fixed, A_flat, B_flat, bias.reshape(1, D))
    return out, loss[0, 0]


def kernel(x, Wg1, bg1, Wg2, bg2, weight_fixed, A_all, B_all, bias):
    A_flat = A_all.reshape(E * R, D)
    B_flat = B_all.transpose(0, 2, 1).reshape(E * R, D)
    return _run(x, Wg1, bg1, Wg2, bg2, weight_fixed, A_flat, B_flat, bias)


# speedup vs baseline: 1.9865x; 1.1014x over previous
import jax
import jax.numpy as jnp
from jax.experimental import pallas as pl
from jax.experimental.pallas import tpu as pltpu

N = 2048
D = 1024
E = 8
R = 16
H = 256
K = 2

BN = 1024
GRID = N // BN


def _body(x_ref, wg1_ref, bg1_ref, wg2_ref, bg2_ref, wf_ref, a_ref, b_ref,
          bias_ref, out_ref, loss_ref, acc_m, acc_p):
    step = pl.program_id(0)
    xb = x_ref[...]

    h = jax.lax.dot_general(xb, wg1_ref[...], (((1,), (1,)), ((), ())),
                            preferred_element_type=jnp.float32)
    h = jnp.maximum(h + bg1_ref[...], 0.0)
    logits = jax.lax.dot_general(wg2_ref[...], h, (((1,), (1,)), ((), ())),
                                 preferred_element_type=jnp.float32)
    logits = logits + bg2_ref[...]

    neg = jnp.float32(-jnp.inf)
    m1 = jnp.full((1, BN), neg, jnp.float32)
    i1 = jnp.zeros((1, BN), jnp.int32)
    for e in range(E):
        v = logits[e:e + 1, :]
        take = v > m1
        m1 = jnp.where(take, v, m1)
        i1 = jnp.where(take, e, i1)
    m2 = jnp.full((1, BN), neg, jnp.float32)
    i2 = jnp.zeros((1, BN), jnp.int32)
    for e in range(E):
        v = logits[e:e + 1, :]
        take = (v > m2) & (i1 != e)
        m2 = jnp.where(take, v, m2)
        i2 = jnp.where(take, e, i2)

    a = jnp.exp(m2 - m1)
    g1 = 1.0 / (1.0 + a)
    g2 = 1.0 - g1

    eiota = jax.lax.broadcasted_iota(jnp.int32, (E, BN), 0)
    sel1 = eiota == i1
    sel2 = eiota == i2
    wt = jnp.where(sel1, g1, 0.0) + jnp.where(sel2, g2, 0.0)
    maskt = (sel1 | sel2).astype(jnp.float32)

    ex = jnp.exp(logits - m1)
    probst = ex / jnp.sum(ex, axis=0, keepdims=True)

    of = jax.lax.dot_general(xb, wf_ref[...], (((1,), (1,)), ((), ())),
                             preferred_element_type=jnp.float32,
                             precision=jax.lax.Precision.DEFAULT)
    xa = jax.lax.dot_general(xb, a_ref[...], (((1,), (1,)), ((), ())),
                             preferred_element_type=jnp.float32,
                             precision=jax.lax.Precision.DEFAULT)

    rows = jax.lax.broadcasted_iota(jnp.int32, (E, E * R), 0)
    cols = jax.lax.broadcasted_iota(jnp.int32, (E, E * R), 1)
    sel = (cols // R == rows).astype(jnp.float32)
    w_wide = jax.lax.dot_general(wt, sel, (((0,), (0,)), ((), ())),
                                 preferred_element_type=jnp.float32)

    eo = jax.lax.dot_general(xa * w_wide, b_ref[...], (((1,), (0,)), ((), ())),
                             preferred_element_type=jnp.float32,
                             precision=jax.lax.Precision.DEFAULT)
    out_ref[...] = of + eo + bias_ref[...]

    @pl.when(step == 0)
    def _():
        acc_m[...] = maskt
        acc_p[...] = probst

    @pl.when(step > 0)
    def _():
        acc_m[...] += maskt
        acc_p[...] += probst

    @pl.when(step == GRID - 1)
    def _():
        am = jnp.sum(acc_m[...], axis=1, keepdims=True)
        ap = jnp.sum(acc_p[...], axis=1, keepdims=True)
        loss = (E / (N * N)) * jnp.sum(am * ap, keepdims=True)
        loss_ref[...] = loss.reshape(1, 1)


@jax.jit
def _run(x, Wg1, bg1, Wg2, bg2, weight_fixed, A_flat, B_flat, bias):
    full = lambda s: pl.BlockSpec(s, lambda i: (0, 0))
    out, loss = pl.pallas_call(
        _body,
        grid=(GRID,),
        in_specs=[
            pl.BlockSpec((BN, D), lambda i: (i, 0)),
            full((H, D)),
            full((1, H)),
            full((E, H)),
            full((E, 1)),
            full((D, D)),
            full((E * R, D)),
            full((E * R, D)),
            full((1, D)),
        ],
        out_specs=[
            pl.BlockSpec((BN, D), lambda i: (i, 0)),
            full((1, 1)),
        ],
        out_shape=[
            jax.ShapeDtypeStruct((N, D), jnp.float32),
            jax.ShapeDtypeStruct((1, 1), jnp.float32),
        ],
        scratch_shapes=[
            pltpu.VMEM((E, BN), jnp.float32),
            pltpu.VMEM((E, BN), jnp.float32),
        ],
        compiler_params=pltpu.CompilerParams(
            dimension_semantics=("arbitrary",),
        ),
    )(x, Wg1, bg1.reshape(1, H), Wg2, bg2.reshape(E, 1),
      weight_fixed, A_flat, B_flat, bias.reshape(1, D))
    return out, loss[0, 0]


def kernel(x, Wg1, bg1, Wg2, bg2, weight_fixed, A_all, B_all, bias):
    A_flat = A_all.reshape(E * R, D)
    B_flat = B_all.transpose(0, 2, 1).reshape(E * R, D)
    return _run(x, Wg1, bg1, Wg2, bg2, weight_fixed, A_flat, B_flat, bias)


# speedup vs baseline: 2.0141x; 1.0139x over previous
import jax
import jax.numpy as jnp
from jax.experimental import pallas as pl
from jax.experimental.pallas import tpu as pltpu

N = 2048
D = 1024
E = 8
R = 16
H = 256
K = 2

BN = 512
GRID = N // BN


def _body(x_ref, wg1_ref, bg1_ref, wg2_ref, bg2_ref, wf_ref, a_ref, b_ref,
          bias_ref, out_ref, loss_ref, acc_m, acc_p):
    step = pl.program_id(0)
    xb = x_ref[...]

    h = jax.lax.dot_general(xb, wg1_ref[...], (((1,), (1,)), ((), ())),
                            preferred_element_type=jnp.float32)
    h = jnp.maximum(h + bg1_ref[...], 0.0)
    logits = jax.lax.dot_general(wg2_ref[...], h, (((1,), (1,)), ((), ())),
                                 preferred_element_type=jnp.float32)
    logits = logits + bg2_ref[...]

    neg = jnp.float32(-jnp.inf)
    m1 = jnp.full((1, BN), neg, jnp.float32)
    i1 = jnp.zeros((1, BN), jnp.int32)
    for e in range(E):
        v = logits[e:e + 1, :]
        take = v > m1
        m1 = jnp.where(take, v, m1)
        i1 = jnp.where(take, e, i1)
    m2 = jnp.full((1, BN), neg, jnp.float32)
    i2 = jnp.zeros((1, BN), jnp.int32)
    for e in range(E):
        v = logits[e:e + 1, :]
        take = (v > m2) & (i1 != e)
        m2 = jnp.where(take, v, m2)
        i2 = jnp.where(take, e, i2)

    a = jnp.exp(m2 - m1)
    g1 = 1.0 / (1.0 + a)
    g2 = 1.0 - g1

    eiota = jax.lax.broadcasted_iota(jnp.int32, (E, BN), 0)
    sel1 = eiota == i1
    sel2 = eiota == i2
    wt = jnp.where(sel1, g1, 0.0) + jnp.where(sel2, g2, 0.0)
    maskt = (sel1 | sel2).astype(jnp.float32)

    ex = jnp.exp(logits - m1)
    probst = ex / jnp.sum(ex, axis=0, keepdims=True)

    of = jax.lax.dot_general(xb, wf_ref[...], (((1,), (1,)), ((), ())),
                             preferred_element_type=jnp.float32,
                             precision=jax.lax.Precision.DEFAULT)
    xa = jax.lax.dot_general(xb, a_ref[...], (((1,), (1,)), ((), ())),
                             preferred_element_type=jnp.float32,
                             precision=jax.lax.Precision.DEFAULT)

    rows = jax.lax.broadcasted_iota(jnp.int32, (E, E * R), 0)
    cols = jax.lax.broadcasted_iota(jnp.int32, (E, E * R), 1)
    sel = (cols // R == rows).astype(jnp.float32)
    w_wide = jax.lax.dot_general(wt, sel, (((0,), (0,)), ((), ())),
                                 preferred_element_type=jnp.float32)

    eo = jax.lax.dot_general(xa * w_wide, b_ref[...], (((1,), (0,)), ((), ())),
                             preferred_element_type=jnp.float32,
                             precision=jax.lax.Precision.DEFAULT)
    out_ref[...] = of + eo + bias_ref[...]

    @pl.when(step == 0)
    def _():
        acc_m[...] = maskt
        acc_p[...] = probst

    @pl.when(step > 0)
    def _():
        acc_m[...] += maskt
        acc_p[...] += probst

    @pl.when(step == GRID - 1)
    def _():
        am = jnp.sum(acc_m[...], axis=1, keepdims=True)
        ap = jnp.sum(acc_p[...], axis=1, keepdims=True)
        loss = (E / (N * N)) * jnp.sum(am * ap, keepdims=True)
        loss_ref[...] = loss.reshape(1, 1)


@jax.jit
def _run(x, Wg1, bg1, Wg2, bg2, weight_fixed, A_flat, B_flat, bias):
    full = lambda s: pl.BlockSpec(s, lambda i: (0, 0))
    out, loss = pl.pallas_call(
        _body,
        grid=(GRID,),
        in_specs=[
            pl.BlockSpec((BN, D), lambda i: (i, 0)),
            full((H, D)),
            full((1, H)),
            full((E, H)),
            full((E, 1)),
            full((D, D)),
            full((E * R, D)),
            full((E * R, D)),
            full((1, D)),
        ],
        out_specs=[
            pl.BlockSpec((BN, D), lambda i: (i, 0)),
            full((1, 1)),
        ],
        out_shape=[
            jax.ShapeDtypeStruct((N, D), jnp.float32),
            jax.ShapeDtypeStruct((1, 1), jnp.float32),
        ],
        scratch_shapes=[
            pltpu.VMEM((E, BN), jnp.float32),
            pltpu.VMEM((E, BN), jnp.float32),
        ],
        compiler_params=pltpu.CompilerParams(
            dimension_semantics=("arbitrary",),
        ),
    )(x, Wg1, bg1.reshape(1, H), Wg2, bg2.reshape(E, 1),
      weight_fixed, A_flat, B_flat, bias.reshape(1, D))
    return out, loss[0, 0]


def kernel(x, Wg1, bg1, Wg2, bg2, weight_fixed, A_all, B_all, bias):
    A_flat = A_all.reshape(E * R, D)
    B_flat = B_all.transpose(0, 2, 1).reshape(E * R, D)
    return _run(x, Wg1, bg1, Wg2, bg2, weight_fixed, A_flat, B_flat, bias)
